# baseline (speedup 1.0000x reference)
import functools

import jax
import jax.numpy as jnp
from jax import lax
from jax.experimental import pallas as pl
from jax.experimental.pallas import tpu as pltpu
from jax.experimental.pallas import tpu_sc as plsc

D = 64
LANES = 16
MS_ROWS = 13 * 5
DW_ROWS = 32 * 8
H_ROWS = 25


@functools.cache
def _build(B, L):
    info = plsc.get_sparse_core_info()
    NC, NS = info.num_cores, info.num_subcores
    NW = NC * NS
    seq_per_w = B // NW
    assert seq_per_w * NW == B and seq_per_w % 2 == 0
    n_pairs = seq_per_w // 2
    n_full = L // LANES
    goffs = [g * LANES for g in range(n_full)]
    if L % LANES:
        goffs.append(L - LANES)
    n_groups = len(goffs)
    tail_off = goffs[-1]

    mesh = plsc.VectorSubcoreMesh(core_axis_name="c", subcore_axis_name="s")

    @functools.partial(
        pl.kernel,
        mesh=mesh,
        out_type=jax.ShapeDtypeStruct((B, L, D), jnp.float32),
        scratch_types=(
            [pltpu.VMEM((L,), jnp.int32) for _ in range(12)]
            + [pltpu.VMEM((L,), jnp.int32) for _ in range(10)]
            + [pltpu.VMEM((L, 2 * D), jnp.float32) for _ in range(2)]
            + [pltpu.VMEM((L, D), jnp.float32) for _ in range(2)]
            + [pltpu.VMEM((MS_ROWS * D,), jnp.float32),
               pltpu.VMEM((DW_ROWS * D,), jnp.float32),
               pltpu.VMEM((H_ROWS * D,), jnp.float32)]
            + [pltpu.SemaphoreType.DMA for _ in range(6)]
        ),
    )
    def k(y_i, m_i, d_i, s_i, h_i, w_i,
          y_t, m_t, d_t, s_t, h_t, w_t,
          out,
          sy0, sm0, sd0, ss0, sh0, sw0,
          sy1, sm1, sd1, ss1, sh1, sw1,
          iyg0, x00, x10, x20, x30,
          iyg1, x01, x11, x21, x31,
          by0, by1, ob0, ob1,
          cms, cdw, ch,
          semi0, semi1, semy0, semy1, semo0, semo1):
        wid = lax.axis_index("s") * NC + lax.axis_index("c")
        wbase = wid * seq_per_w

        sets = (
            ((sy0, sm0, sd0, ss0, sh0, sw0),
             iyg0, x00, (x10, x20, x30), by0, ob0, semi0, semy0, semo0),
            ((sy1, sm1, sd1, ss1, sh1, sw1),
             iyg1, x01, (x11, x21, x31), by1, ob1, semi1, semy1, semo1),
        )
        idx_hbm = (y_i, m_i, d_i, s_i, h_i, w_i)

        pltpu.sync_copy(m_t, ch.at[pl.ds(0, 13 * D)])
        pltpu.sync_copy(s_t, ch.at[pl.ds(13 * D, 5 * D)])
        pltpu.sync_copy(d_t, cms.at[pl.ds(0, 32 * D)])
        pltpu.sync_copy(w_t, cms.at[pl.ds(32 * D, 8 * D)])

        @plsc.parallel_loop(0, DW_ROWS, unroll=2)
        def _(r):
            d = r // 8
            w = r - d * 8
            for c in range(D // LANES):
                cdw[pl.ds(r * D + c * LANES, LANES)] = (
                    cms[pl.ds(d * D + c * LANES, LANES)]
                    + cms[pl.ds(32 * D + w * D + c * LANES, LANES)])

        @plsc.parallel_loop(0, MS_ROWS, unroll=2)
        def _(r):
            m = r // 5
            s = r - m * 5
            for c in range(D // LANES):
                cms[pl.ds(r * D + c * LANES, LANES)] = (
                    ch[pl.ds(m * D + c * LANES, LANES)]
                    + ch[pl.ds(13 * D + s * D + c * LANES, LANES)])

        pltpu.sync_copy(h_t, ch)

        def issue_idx(chunk, st):
            for hbm, vref in zip(idx_hbm, st[0]):
                pltpu.async_copy(hbm.at[wbase + chunk], vref, st[6])

        def wait_idx(st):
            for hbm, vref in zip(idx_hbm, st[0]):
                pltpu.make_async_copy(hbm.at[wbase], vref, st[6]).wait()

        def prep(st):
            sy, iyg, x0 = st[0][0], st[1], st[2]
            for off in goffs:
                sl = pl.ds(off, LANES)
                yv = sy[sl]
                iyg[sl] = yv >> 1
                x0[sl] = (yv & 1) << 6

        def issue_gathers(st):
            iyg, by, semy = st[1], st[4], st[7]
            pltpu.async_copy(y_t.at[iyg.at[pl.ds(0, 128)]],
                             by.at[pl.ds(0, 128)], semy)
            pltpu.async_copy(y_t.at[iyg.at[pl.ds(128, L - 128)]],
                             by.at[pl.ds(128, L - 128)], semy)

        def wait_gathers(st):
            iyg, by, semy = st[1], st[4], st[7]
            pltpu.make_async_copy(y_t.at[iyg.at[pl.ds(0, 128)]],
                                  by.at[pl.ds(0, 128)], semy).wait()
            pltpu.make_async_copy(y_t.at[iyg.at[pl.ds(128, L - 128)]],
                                  by.at[pl.ds(128, L - 128)], semy).wait()

        def wait_out(st):
            pltpu.make_async_copy(st[5], out.at[wbase], st[8]).wait()

        issue_idx(0, sets[0])
        issue_idx(1, sets[1])
        wait_idx(sets[0])
        prep(sets[0])
        issue_gathers(sets[0])

        def pair_body(p, carry):
            for b in range(2):
                st = sets[b]
                st2 = sets[1 - b]
                idxv = st[0]
                x0, (x1, x2, x3), by, ob, semo = st[2], st[3], st[4], st[5], st[8]
                chunk = p * 2 + b
                bb = wbase + chunk

                wait_gathers(st)

                for off in goffs:
                    sl = pl.ds(off, LANES)
                    mv = idxv[1][sl]
                    dv = idxv[2][sl]
                    sv = idxv[3][sl]
                    hv = idxv[4][sl]
                    wv = idxv[5][sl]
                    x1[sl] = (mv * 5 + sv) * D
                    x2[sl] = ((dv << 3) + wv) * D
                    x3[sl] = hv * D

                @pl.when(p < n_pairs - 1)
                def _():
                    issue_idx(chunk + 2, st)

                if b == 0:
                    @pl.when(p > 0)
                    def _():
                        wait_out(st2)
                    wait_idx(st2)
                    prep(st2)
                    issue_gathers(st2)
                else:
                    @pl.when(p < n_pairs - 1)
                    def _():
                        wait_out(st2)
                        wait_idx(st2)
                        prep(st2)
                        issue_gathers(st2)

                @plsc.parallel_loop(0, n_groups, unroll=2)
                def _(g):
                    off = lax.min(g * LANES, tail_off)
                    x0v = x0[pl.ds(off, LANES)]
                    i1v = x1[pl.ds(off, LANES)]
                    i2v = x2[pl.ds(off, LANES)]
                    i3v = x3[pl.ds(off, LANES)]
                    for j in range(LANES):
                        p0 = x0v[j]
                        i1 = i1v[j]
                        i2 = i2v[j]
                        i3 = i3v[j]
                        r = off + j
                        for c in range(D // LANES):
                            sl = pl.ds(c * LANES, LANES)
                            ob[r, sl] = (
                                by[r, pl.ds(p0 + c * LANES, LANES)]
                                + cms[pl.ds(i1 + c * LANES, LANES)]
                                + cdw[pl.ds(i2 + c * LANES, LANES)]
                                + ch[pl.ds(i3 + c * LANES, LANES)])

                pltpu.async_copy(ob, out.at[bb], semo)
            return carry
        lax.fori_loop(0, n_pairs, pair_body, 0)

        wait_out(sets[0])
        wait_out(sets[1])

    return k


def kernel(years, months, days, seasons, hours, dayofweek,
           years_emb, months_emb, days_emb, seasons_emb, hours_emb,
           dayofweek_emb):
    B, L = years.shape
    idx = [a.astype(jnp.int32)
           for a in (years, months, days, seasons, hours, dayofweek)]
    y2 = years_emb.reshape(years_emb.shape[0] // 2, 2 * D)
    small = [t.reshape(-1)
             for t in (months_emb, days_emb, seasons_emb, hours_emb,
                       dayofweek_emb)]
    return _build(B, L)(*idx, y2, *small)
